# baseline (speedup 1.0000x reference)
import functools

import jax
import jax.numpy as jnp
from jax import lax
from jax.experimental import pallas as pl
from jax.experimental.pallas import tpu as pltpu
from jax.experimental.pallas import tpu_sc as plsc

_NC = 2
_NS = 16
_NW = _NC * _NS
_LANES = 16


def _sc_dispatch_gather(inputs, tok, outidx, n_pad):
    _, d_in = inputs.shape
    R = tok.shape[0]
    rpw = R // _NW
    chunk = 32
    n_chunks = rpw // chunk
    mesh = plsc.VectorSubcoreMesh(core_axis_name="c", subcore_axis_name="s")

    @functools.partial(
        pl.kernel,
        out_type=jax.ShapeDtypeStruct((n_pad, d_in), jnp.float32),
        mesh=mesh,
        scratch_types=(
            pltpu.VMEM((rpw,), jnp.int32),
            pltpu.VMEM((n_chunks, chunk), jnp.int32),
            pltpu.VMEM((3, chunk, d_in), jnp.float32),
            pltpu.SemaphoreType.DMA,
            pltpu.SemaphoreType.DMA,
            pltpu.SemaphoreType.DMA,
            pltpu.SemaphoreType.DMA,
            pltpu.SemaphoreType.DMA,
            pltpu.SemaphoreType.DMA,
        ),
    )
    def run(inputs_hbm, tok_hbm, outidx_hbm, xpad_hbm, idx_v, oidx_v, rows_v,
            gsem0, gsem1, gsem2, ssem0, ssem1, ssem2):
        wid = lax.axis_index("s") * _NC + lax.axis_index("c")
        base = wid * rpw
        pltpu.sync_copy(tok_hbm.at[pl.ds(base, rpw)], idx_v)
        pltpu.sync_copy(outidx_hbm.at[wid], oidx_v)
        gsems = (gsem0, gsem1, gsem2)
        ssems = (ssem0, ssem1, ssem2)

        def start_gather(c):
            b = c % 3
            return pltpu.async_copy(
                inputs_hbm.at[idx_v.at[pl.ds(c * chunk, chunk)]],
                rows_v.at[b], gsems[b])

        gh = {0: start_gather(0)}
        sh = {}
        for c in range(n_chunks):
            b = c % 3
            if c + 1 < n_chunks:
                if c - 2 >= 0:
                    sh.pop(c - 2).wait()
                gh[c + 1] = start_gather(c + 1)
            gh.pop(c).wait()
            sh[c] = pltpu.async_copy(
                rows_v.at[b], xpad_hbm.at[oidx_v.at[c]], ssems[b])
        for c in sorted(sh):
            sh.pop(c).wait()

    return run(inputs, tok, outidx)


def _sc_combine(y, inv, gates_flat, n_tokens, kk):
    R, d_out = y.shape
    tpw = n_tokens // _NW
    ct = _LANES // kk
    n_chunks = tpw // ct
    vregs = d_out // _LANES
    mesh = plsc.VectorSubcoreMesh(core_axis_name="c", subcore_axis_name="s")

    @functools.partial(
        pl.kernel,
        out_type=jax.ShapeDtypeStruct((n_tokens, d_out), jnp.float32),
        mesh=mesh,
        scratch_types=(
            pltpu.VMEM((tpw * kk,), jnp.int32),
            pltpu.VMEM((tpw * kk,), jnp.float32),
            pltpu.VMEM((2, ct * kk, d_out), jnp.float32),
            pltpu.VMEM((2, ct, d_out), jnp.float32),
            pltpu.SemaphoreType.DMA,
            pltpu.SemaphoreType.DMA,
            pltpu.SemaphoreType.DMA,
            pltpu.SemaphoreType.DMA,
        ),
    )
    def run(y_hbm, inv_hbm, gates_hbm, res_hbm, idx_v, g_v, ybuf_v, obuf_v,
            gsem0, gsem1, ssem0, ssem1):
        wid = lax.axis_index("s") * _NC + lax.axis_index("c")
        tbase = wid * tpw
        pltpu.sync_copy(inv_hbm.at[pl.ds(tbase * kk, tpw * kk)], idx_v)
        pltpu.sync_copy(gates_hbm.at[pl.ds(tbase * kk, tpw * kk)], g_v)
        gsems = (gsem0, gsem1)
        ssems = (ssem0, ssem1)

        def start_gather(c):
            b = c % 2
            return pltpu.async_copy(
                y_hbm.at[idx_v.at[pl.ds(c * ct * kk, ct * kk)]],
                ybuf_v.at[b], gsems[b])

        gh = {0: start_gather(0)}
        sh = {}
        for c in range(n_chunks):
            b = c % 2
            if c + 1 < n_chunks:
                gh[c + 1] = start_gather(c + 1)
            gh.pop(c).wait()
            if c >= 2:
                sh.pop(c - 2).wait()
            greg = g_v[pl.ds(c * ct * kk, _LANES)]
            gs = [greg[i] for i in range(ct * kk)]

            def body(j, _):
                for t in range(ct):
                    acc = gs[t * kk] * ybuf_v[b, t * kk,
                                              pl.ds(j * _LANES, _LANES)]
                    for jj in range(1, kk):
                        acc = acc + gs[t * kk + jj] * ybuf_v[
                            b, t * kk + jj, pl.ds(j * _LANES, _LANES)]
                    obuf_v[b, t, pl.ds(j * _LANES, _LANES)] = acc
                return 0

            lax.fori_loop(0, vregs, body, 0)
            sh[c] = pltpu.async_copy(
                obuf_v.at[b], res_hbm.at[pl.ds(tbase + c * ct, ct)], ssems[b])
        for c in sorted(sh):
            sh.pop(c).wait()

    return run(y, inv, gates_flat)


def _padded_matmul(x_pad, weight, blk_expert, *, br):
    P, d_in = x_pad.shape
    E, d_out, _ = weight.shape
    nb = P // br

    def body(eid, x_ref, w_ref, y_ref):
        del eid
        y_ref[...] = jax.lax.dot_general(
            x_ref[...], w_ref[0], (((1,), (1,)), ((), ())),
            preferred_element_type=jnp.float32)

    grid_spec = pltpu.PrefetchScalarGridSpec(
        num_scalar_prefetch=1,
        grid=(nb,),
        in_specs=[
            pl.BlockSpec((br, d_in), lambda i, eid: (i, 0)),
            pl.BlockSpec((1, d_out, d_in), lambda i, eid: (eid[i], 0, 0)),
        ],
        out_specs=pl.BlockSpec((br, d_out), lambda i, eid: (i, 0)),
    )
    return pl.pallas_call(
        body,
        grid_spec=grid_spec,
        out_shape=jax.ShapeDtypeStruct((P, d_out), jnp.float32),
        compiler_params=pltpu.CompilerParams(
            dimension_semantics=("arbitrary",)),
    )(blk_expert, x_pad, weight)


def kernel(inputs, weight, gates, k, sorted_expert_idxs, sorted_scattered_idxs,
           expert_offsets):
    del k, sorted_expert_idxs
    n, kk = gates.shape
    E = weight.shape[0]
    br = 256
    ssi = sorted_scattered_idxs.astype(jnp.int32)
    ends = expert_offsets.astype(jnp.int32)
    starts = jnp.concatenate([jnp.zeros((1,), jnp.int32), ends[:-1]])
    counts = ends - starts
    n_rows = n * kk
    n_pad = n_rows + E * br

    pcounts = ((counts + br - 1) // br) * br
    pends = jnp.cumsum(pcounts)
    pstarts = pends - pcounts
    padoff = pstarts - starts

    r = jnp.arange(n_rows, dtype=jnp.int32)
    e_r = jnp.minimum(
        jnp.searchsorted(ends, r, side="right").astype(jnp.int32), E - 1)
    outidx = (r + padoff[e_r]).reshape(_NW, -1, 32)
    tok = ssi // kk

    blk_expert = jnp.minimum(
        jnp.searchsorted(pends, jnp.arange(n_pad // br, dtype=jnp.int32) * br,
                         side="right").astype(jnp.int32), E - 1)

    x_pad = _sc_dispatch_gather(inputs, tok, outidx, n_pad)
    y_pad = _padded_matmul(x_pad, weight, blk_expert, br=br)

    inv = jnp.zeros((n_rows,), jnp.int32).at[ssi].set(
        jnp.arange(n_rows, dtype=jnp.int32))
    e_s = jnp.minimum(
        jnp.searchsorted(ends, inv, side="right").astype(jnp.int32), E - 1)
    inv_pad = inv + padoff[e_s]
    return _sc_combine(y_pad, inv_pad, gates.reshape(-1), n, kk)


# speedup vs baseline: 1.3497x; 1.3497x over previous
import functools

import jax
import jax.numpy as jnp
from jax import lax
from jax.experimental import pallas as pl
from jax.experimental.pallas import tpu as pltpu
from jax.experimental.pallas import tpu_sc as plsc

_NC = 2
_NS = 16
_NW = _NC * _NS
_LANES = 16


def _sc_dispatch_gather(inputs, tok, outidx, n_pad):
    _, d_in = inputs.shape
    R = tok.shape[0]
    rpw = R // _NW
    chunk = 32
    n_chunks = rpw // chunk
    mesh = plsc.VectorSubcoreMesh(core_axis_name="c", subcore_axis_name="s")

    @functools.partial(
        pl.kernel,
        out_type=jax.ShapeDtypeStruct((n_pad, d_in), jnp.float32),
        mesh=mesh,
        scratch_types=(
            pltpu.VMEM((rpw,), jnp.int32),
            pltpu.VMEM((n_chunks, chunk), jnp.int32),
            pltpu.VMEM((3, chunk, d_in), jnp.float32),
            pltpu.SemaphoreType.DMA,
            pltpu.SemaphoreType.DMA,
            pltpu.SemaphoreType.DMA,
            pltpu.SemaphoreType.DMA,
            pltpu.SemaphoreType.DMA,
            pltpu.SemaphoreType.DMA,
        ),
    )
    def run(inputs_hbm, tok_hbm, outidx_hbm, xpad_hbm, idx_v, oidx_v, rows_v,
            gsem0, gsem1, gsem2, ssem0, ssem1, ssem2):
        wid = lax.axis_index("s") * _NC + lax.axis_index("c")
        base = wid * rpw
        pltpu.sync_copy(tok_hbm.at[pl.ds(base, rpw)], idx_v)
        pltpu.sync_copy(outidx_hbm.at[wid], oidx_v)
        gsems = (gsem0, gsem1, gsem2)
        ssems = (ssem0, ssem1, ssem2)

        def start_gather(c):
            b = c % 3
            return pltpu.async_copy(
                inputs_hbm.at[idx_v.at[pl.ds(c * chunk, chunk)]],
                rows_v.at[b], gsems[b])

        gh = {0: start_gather(0)}
        sh = {}
        for c in range(n_chunks):
            b = c % 3
            if c + 1 < n_chunks:
                if c - 2 >= 0:
                    sh.pop(c - 2).wait()
                gh[c + 1] = start_gather(c + 1)
            gh.pop(c).wait()
            sh[c] = pltpu.async_copy(
                rows_v.at[b], xpad_hbm.at[oidx_v.at[c]], ssems[b])
        for c in sorted(sh):
            sh.pop(c).wait()

    return run(inputs, tok, outidx)


def _sc_combine(y, inv, gates_flat, n_tokens, kk):
    R, d_out = y.shape
    tpw = n_tokens // _NW
    ct = _LANES // kk
    n_chunks = tpw // ct
    vregs = d_out // _LANES
    mesh = plsc.VectorSubcoreMesh(core_axis_name="c", subcore_axis_name="s")

    @functools.partial(
        pl.kernel,
        out_type=jax.ShapeDtypeStruct((n_tokens, d_out), jnp.float32),
        mesh=mesh,
        scratch_types=(
            pltpu.VMEM((tpw * kk,), jnp.int32),
            pltpu.VMEM((tpw * kk,), jnp.float32),
            pltpu.VMEM((2, ct * kk, d_out), jnp.float32),
            pltpu.VMEM((2, ct, d_out), jnp.float32),
            pltpu.SemaphoreType.DMA,
            pltpu.SemaphoreType.DMA,
            pltpu.SemaphoreType.DMA,
            pltpu.SemaphoreType.DMA,
        ),
    )
    def run(y_hbm, inv_hbm, gates_hbm, res_hbm, idx_v, g_v, ybuf_v, obuf_v,
            gsem0, gsem1, ssem0, ssem1):
        wid = lax.axis_index("s") * _NC + lax.axis_index("c")
        tbase = wid * tpw
        pltpu.sync_copy(inv_hbm.at[pl.ds(tbase * kk, tpw * kk)], idx_v)
        pltpu.sync_copy(gates_hbm.at[pl.ds(tbase * kk, tpw * kk)], g_v)
        gsems = (gsem0, gsem1)
        ssems = (ssem0, ssem1)

        def start_gather(c):
            b = c % 2
            return pltpu.async_copy(
                y_hbm.at[idx_v.at[pl.ds(c * ct * kk, ct * kk)]],
                ybuf_v.at[b], gsems[b])

        gh = {0: start_gather(0)}
        sh = {}
        for c in range(n_chunks):
            b = c % 2
            if c + 1 < n_chunks:
                gh[c + 1] = start_gather(c + 1)
            gh.pop(c).wait()
            if c >= 2:
                sh.pop(c - 2).wait()
            greg = g_v[pl.ds(c * ct * kk, _LANES)]
            gs = [greg[i] for i in range(ct * kk)]

            def body(j, _):
                for t in range(ct):
                    acc = gs[t * kk] * ybuf_v[b, t * kk,
                                              pl.ds(j * _LANES, _LANES)]
                    for jj in range(1, kk):
                        acc = acc + gs[t * kk + jj] * ybuf_v[
                            b, t * kk + jj, pl.ds(j * _LANES, _LANES)]
                    obuf_v[b, t, pl.ds(j * _LANES, _LANES)] = acc
                return 0

            lax.fori_loop(0, vregs, body, 0)
            sh[c] = pltpu.async_copy(
                obuf_v.at[b], res_hbm.at[pl.ds(tbase + c * ct, ct)], ssems[b])
        for c in sorted(sh):
            sh.pop(c).wait()

    return run(y, inv, gates_flat)


def _padded_matmul(x_pad, weight, blk_expert, *, br):
    P, d_in = x_pad.shape
    E, d_out, _ = weight.shape
    nb = P // br

    def body(eid, x_ref, w_ref, y_ref):
        del eid
        y_ref[...] = jax.lax.dot_general(
            x_ref[...], w_ref[0], (((1,), (1,)), ((), ())),
            preferred_element_type=jnp.float32)

    grid_spec = pltpu.PrefetchScalarGridSpec(
        num_scalar_prefetch=1,
        grid=(nb,),
        in_specs=[
            pl.BlockSpec((br, d_in), lambda i, eid: (i, 0)),
            pl.BlockSpec((1, d_out, d_in), lambda i, eid: (eid[i], 0, 0)),
        ],
        out_specs=pl.BlockSpec((br, d_out), lambda i, eid: (i, 0)),
    )
    return pl.pallas_call(
        body,
        grid_spec=grid_spec,
        out_shape=jax.ShapeDtypeStruct((P, d_out), jnp.float32),
        compiler_params=pltpu.CompilerParams(
            dimension_semantics=("arbitrary",)),
    )(blk_expert, x_pad, weight)


def kernel(inputs, weight, gates, k, sorted_expert_idxs, sorted_scattered_idxs,
           expert_offsets):
    del k, sorted_expert_idxs
    n, kk = gates.shape
    E = weight.shape[0]
    br = 256
    ssi = sorted_scattered_idxs.astype(jnp.int32)
    ends = expert_offsets.astype(jnp.int32)
    starts = jnp.concatenate([jnp.zeros((1,), jnp.int32), ends[:-1]])
    counts = ends - starts
    n_rows = n * kk
    n_pad = n_rows + E * br

    pcounts = ((counts + br - 1) // br) * br
    pends = jnp.cumsum(pcounts)
    pstarts = pends - pcounts
    padoff = pstarts - starts

    r = jnp.arange(n_rows, dtype=jnp.int32)
    e_r = jnp.minimum(
        (ends[None, :] <= r[:, None]).sum(axis=1).astype(jnp.int32), E - 1)
    outidx = (r + padoff[e_r]).reshape(_NW, -1, 32)
    tok = ssi // kk

    bstart = jnp.arange(n_pad // br, dtype=jnp.int32) * br
    blk_expert = jnp.minimum(
        (pends[None, :] <= bstart[:, None]).sum(axis=1).astype(jnp.int32),
        E - 1)

    x_pad = _sc_dispatch_gather(inputs, tok, outidx, n_pad)
    y_pad = _padded_matmul(x_pad, weight, blk_expert, br=br)

    inv = jnp.zeros((n_rows,), jnp.int32).at[ssi].set(
        jnp.arange(n_rows, dtype=jnp.int32))
    e_s = jnp.minimum(
        (ends[None, :] <= inv[:, None]).sum(axis=1).astype(jnp.int32), E - 1)
    inv_pad = inv + padoff[e_s]
    return _sc_combine(y_pad, inv_pad, gates.reshape(-1), n, kk)


# speedup vs baseline: 1.3510x; 1.0010x over previous
import functools

import jax
import jax.numpy as jnp
from jax import lax
from jax.experimental import pallas as pl
from jax.experimental.pallas import tpu as pltpu
from jax.experimental.pallas import tpu_sc as plsc

_NC = 2
_NS = 16
_NW = _NC * _NS
_LANES = 16


def _sc_dispatch_gather(inputs, tok, outidx, n_pad):
    _, d_in = inputs.shape
    R = tok.shape[0]
    rpw = R // _NW
    chunk = 32
    n_chunks = rpw // chunk
    mesh = plsc.VectorSubcoreMesh(core_axis_name="c", subcore_axis_name="s")

    @functools.partial(
        pl.kernel,
        out_type=jax.ShapeDtypeStruct((n_pad, d_in), jnp.float32),
        mesh=mesh,
        scratch_types=(
            pltpu.VMEM((rpw,), jnp.int32),
            pltpu.VMEM((n_chunks, chunk), jnp.int32),
            pltpu.VMEM((3, chunk, d_in), jnp.float32),
            pltpu.SemaphoreType.DMA,
            pltpu.SemaphoreType.DMA,
            pltpu.SemaphoreType.DMA,
            pltpu.SemaphoreType.DMA,
            pltpu.SemaphoreType.DMA,
            pltpu.SemaphoreType.DMA,
        ),
    )
    def run(inputs_hbm, tok_hbm, outidx_hbm, xpad_hbm, idx_v, oidx_v, rows_v,
            gsem0, gsem1, gsem2, ssem0, ssem1, ssem2):
        wid = lax.axis_index("s") * _NC + lax.axis_index("c")
        base = wid * rpw
        pltpu.sync_copy(tok_hbm.at[pl.ds(base, rpw)], idx_v)
        pltpu.sync_copy(outidx_hbm.at[wid], oidx_v)
        gsems = (gsem0, gsem1, gsem2)
        ssems = (ssem0, ssem1, ssem2)

        def start_gather(c):
            b = c % 3
            return pltpu.async_copy(
                inputs_hbm.at[idx_v.at[pl.ds(c * chunk, chunk)]],
                rows_v.at[b], gsems[b])

        gh = {0: start_gather(0)}
        sh = {}
        for c in range(n_chunks):
            b = c % 3
            if c + 1 < n_chunks:
                if c - 2 >= 0:
                    sh.pop(c - 2).wait()
                gh[c + 1] = start_gather(c + 1)
            gh.pop(c).wait()
            sh[c] = pltpu.async_copy(
                rows_v.at[b], xpad_hbm.at[oidx_v.at[c]], ssems[b])
        for c in sorted(sh):
            sh.pop(c).wait()

    return run(inputs, tok, outidx)


def _sc_combine(y, inv, gates_flat, n_tokens, kk):
    R, d_out = y.shape
    tpw = n_tokens // _NW
    ct = _LANES // kk
    n_chunks = tpw // ct
    vregs = d_out // _LANES
    mesh = plsc.VectorSubcoreMesh(core_axis_name="c", subcore_axis_name="s")

    @functools.partial(
        pl.kernel,
        out_type=jax.ShapeDtypeStruct((n_tokens, d_out), jnp.float32),
        mesh=mesh,
        scratch_types=(
            pltpu.VMEM((tpw * kk,), jnp.int32),
            pltpu.VMEM((tpw * kk,), jnp.float32),
            pltpu.VMEM((2, ct * kk, d_out), jnp.float32),
            pltpu.VMEM((2, ct, d_out), jnp.float32),
            pltpu.SemaphoreType.DMA,
            pltpu.SemaphoreType.DMA,
            pltpu.SemaphoreType.DMA,
            pltpu.SemaphoreType.DMA,
        ),
    )
    def run(y_hbm, inv_hbm, gates_hbm, res_hbm, idx_v, g_v, ybuf_v, obuf_v,
            gsem0, gsem1, ssem0, ssem1):
        wid = lax.axis_index("s") * _NC + lax.axis_index("c")
        tbase = wid * tpw
        pltpu.sync_copy(inv_hbm.at[pl.ds(tbase * kk, tpw * kk)], idx_v)
        pltpu.sync_copy(gates_hbm.at[pl.ds(tbase * kk, tpw * kk)], g_v)
        gsems = (gsem0, gsem1)
        ssems = (ssem0, ssem1)

        def start_gather(c):
            b = c % 2
            return pltpu.async_copy(
                y_hbm.at[idx_v.at[pl.ds(c * ct * kk, ct * kk)]],
                ybuf_v.at[b], gsems[b])

        gh = {0: start_gather(0)}
        sh = {}
        for c in range(n_chunks):
            b = c % 2
            if c + 1 < n_chunks:
                gh[c + 1] = start_gather(c + 1)
            gh.pop(c).wait()
            if c >= 2:
                sh.pop(c - 2).wait()
            greg = g_v[pl.ds(c * ct * kk, _LANES)]
            gs = [greg[i] for i in range(ct * kk)]

            def body(j, _):
                for t in range(ct):
                    acc = gs[t * kk] * ybuf_v[b, t * kk,
                                              pl.ds(j * _LANES, _LANES)]
                    for jj in range(1, kk):
                        acc = acc + gs[t * kk + jj] * ybuf_v[
                            b, t * kk + jj, pl.ds(j * _LANES, _LANES)]
                    obuf_v[b, t, pl.ds(j * _LANES, _LANES)] = acc
                return 0

            lax.fori_loop(0, vregs, body, 0)
            sh[c] = pltpu.async_copy(
                obuf_v.at[b], res_hbm.at[pl.ds(tbase + c * ct, ct)], ssems[b])
        for c in sorted(sh):
            sh.pop(c).wait()

    return run(y, inv, gates_flat)


def _padded_matmul(x_pad, weight, blk_expert, *, br):
    P, d_in = x_pad.shape
    E, d_out, _ = weight.shape
    nb = P // br

    def body(eid, x_ref, w_ref, y_ref):
        del eid
        y_ref[...] = jax.lax.dot_general(
            x_ref[...], w_ref[0], (((1,), (1,)), ((), ())),
            preferred_element_type=jnp.float32)

    grid_spec = pltpu.PrefetchScalarGridSpec(
        num_scalar_prefetch=1,
        grid=(nb,),
        in_specs=[
            pl.BlockSpec((br, d_in), lambda i, eid: (i, 0)),
            pl.BlockSpec((1, d_out, d_in), lambda i, eid: (eid[i], 0, 0)),
        ],
        out_specs=pl.BlockSpec((br, d_out), lambda i, eid: (i, 0)),
    )
    return pl.pallas_call(
        body,
        grid_spec=grid_spec,
        out_shape=jax.ShapeDtypeStruct((P, d_out), jnp.float32),
        compiler_params=pltpu.CompilerParams(
            dimension_semantics=("arbitrary",)),
    )(blk_expert, x_pad, weight)


def kernel(inputs, weight, gates, k, sorted_expert_idxs, sorted_scattered_idxs,
           expert_offsets):
    del k, sorted_expert_idxs
    n, kk = gates.shape
    E = weight.shape[0]
    br = 256
    ssi = sorted_scattered_idxs.astype(jnp.int32)
    ends = expert_offsets.astype(jnp.int32)
    starts = jnp.concatenate([jnp.zeros((1,), jnp.int32), ends[:-1]])
    counts = ends - starts
    n_rows = n * kk
    n_pad = n_rows + E * br

    pcounts = ((counts + br - 1) // br) * br
    pends = jnp.cumsum(pcounts)
    pstarts = pends - pcounts
    padoff = pstarts - starts

    r = jnp.arange(n_rows, dtype=jnp.int32)
    e_r = jnp.minimum(
        (ends[None, :] <= r[:, None]).sum(axis=1).astype(jnp.int32), E - 1)
    outidx = (r + padoff[e_r]).reshape(_NW, -1, 32)
    tok = ssi // kk

    bstart = jnp.arange(n_pad // br, dtype=jnp.int32) * br
    blk_expert = jnp.minimum(
        (pends[None, :] <= bstart[:, None]).sum(axis=1).astype(jnp.int32),
        E - 1)

    x_pad = _sc_dispatch_gather(inputs, tok, outidx, n_pad)
    y_pad = _padded_matmul(x_pad, weight, blk_expert, br=br)

    inv = jnp.zeros((n_rows,), jnp.int32).at[ssi].set(
        jnp.arange(n_rows, dtype=jnp.int32), unique_indices=True)
    e_s = jnp.minimum(
        (ends[None, :] <= inv[:, None]).sum(axis=1).astype(jnp.int32), E - 1)
    inv_pad = inv + padoff[e_s]
    return _sc_combine(y_pad, inv_pad, gates.reshape(-1), n, kk)
